# baseline (speedup 1.0000x reference)
import jax
import jax.numpy as jnp
from jax import lax
from jax.experimental import pallas as pl
from jax.experimental.pallas import tpu as pltpu
from jax.experimental.pallas import tpu_sc as plsc

N = 10000
D = 128
E = 320000
BLK = 32
NC, NS = 2, 16
NW = NC * NS
NB = 320
NBLK = NB * NW
E_PAD = NBLK * BLK
N_PAD = 10240
ROWS_PER_TILE = N_PAD // NS
GROUPS = D // 16
NBUF = 8
GLEAD = 4
ISL = 8
WCH, WROWS = 5, 128


def _sc_body(x_hbm, src_hbm, dst_hbm, w_hbm, out_hbm,
             src_v, dst_v, w_v, r0, r1, r2, r3, r4, r5, r6, r7, acc,
             g0, g1, g2, g3, g4, g5, g6, g7,
             s0, s1, s2, s3, s4, s5, s6, s7,
             i0, i1, i2, i3, i4, i5, i6, i7):
  rows = (r0, r1, r2, r3, r4, r5, r6, r7)
  gsem = (g0, g1, g2, g3, g4, g5, g6, g7)
  ssem = (s0, s1, s2, s3, s4, s5, s6, s7)
  isem = (i0, i1, i2, i3, i4, i5, i6, i7)
  cid = lax.axis_index("c")
  sid = lax.axis_index("s")
  wid = sid * NC + cid
  e0 = pl.multiple_of(wid * (NB * BLK), BLK)

  def zrow(r, carry):
    for g in range(GROUPS):
      r0[r, pl.ds(g * 16, 16)] = jnp.zeros((16,), jnp.float32)
    return carry
  lax.fori_loop(0, BLK, zrow, 0)
  zbase = sid * ROWS_PER_TILE
  for c in range(ROWS_PER_TILE // BLK):
    pltpu.sync_copy(r0, acc.at[pl.ds(zbase + c * BLK, BLK)])
  plsc.subcore_barrier()

  def ifetch(k, sl):
    off = pl.multiple_of(e0 + k * BLK, BLK)
    pltpu.async_copy(src_hbm.at[pl.ds(off, BLK)], src_v.at[sl], isem[sl])
    pltpu.async_copy(dst_hbm.at[pl.ds(off, BLK)], dst_v.at[sl], isem[sl])
    pltpu.async_copy(w_hbm.at[pl.ds(off, BLK)], w_v.at[sl], isem[sl])

  def iwait(sl):
    pltpu.make_async_copy(src_hbm.at[pl.ds(0, BLK)], src_v.at[sl], isem[sl]).wait()
    pltpu.make_async_copy(dst_hbm.at[pl.ds(0, BLK)], dst_v.at[sl], isem[sl]).wait()
    pltpu.make_async_copy(w_hbm.at[pl.ds(0, BLK)], w_v.at[sl], isem[sl]).wait()

  def gather(sl, b):
    pltpu.async_copy(x_hbm.at[src_v.at[sl]], rows[b], gsem[b])

  def gwait(b):
    pltpu.make_async_copy(x_hbm.at[pl.ds(0, BLK)], rows[b], gsem[b]).wait()

  def scatter(sl, b):
    pltpu.async_copy(rows[b], acc.at[dst_v.at[sl]], ssem[b], add=True)

  def swait(b):
    pltpu.make_async_copy(rows[b], acc.at[pl.ds(0, BLK)], ssem[b]).wait()

  def scale(b, sl):
    rv = rows[b]

    def edge16(j2, inner):
      w16 = w_v[sl, pl.ds(j2 * 16, 16)]
      for t in range(16):
        e = j2 * 16 + t
        w = w16[t]
        for g in range(GROUPS):
          rv[e, pl.ds(g * 16, 16)] = rv[e, pl.ds(g * 16, 16)] * w
      return inner
    lax.fori_loop(0, BLK // 16, edge16, 0)

  for k in range(6):
    ifetch(k, k)
  for k in range(GLEAD):
    iwait(k)
    gather(k, k)

  def super_body(k8, carry):
    for j in range(ISL):
      i = ISL * k8 + j
      bn = (j + GLEAD) % NBUF

      @pl.when(i >= NBUF - GLEAD)
      def _():
        swait(bn)

      @pl.when(i + 6 < NB)
      def _():
        ifetch(i + 6, (j + 6) % ISL)

      @pl.when(i + GLEAD < NB)
      def _():
        iwait((j + GLEAD) % ISL)
        gather((j + GLEAD) % ISL, bn)

      gwait(j)
      scale(j, j)
      scatter(j, j)
    return carry
  lax.fori_loop(0, NB // ISL, super_body, 0)
  for b in range(NBUF - GLEAD, NBUF):
    swait(b)
  plsc.subcore_barrier()

  for c in range(WCH):
    rr = zbase + c * WROWS
    pltpu.sync_copy(acc.at[pl.ds(rr, WROWS)],
                    out_hbm.at[cid, pl.ds(rr, WROWS)])


def _sc_aggregate(x, src_e, dst_e, w_e):
  mesh = plsc.VectorSubcoreMesh(core_axis_name="c", subcore_axis_name="s")
  return pl.kernel(
      _sc_body,
      out_type=jax.ShapeDtypeStruct((NC, N_PAD, D), jnp.float32),
      mesh=mesh,
      scratch_types=(
          [pltpu.VMEM((ISL, BLK), jnp.int32),
           pltpu.VMEM((ISL, BLK), jnp.int32),
           pltpu.VMEM((ISL, BLK), jnp.float32)]
          + [pltpu.VMEM((BLK, D), jnp.float32)] * NBUF
          + [pltpu.VMEM_SHARED((N_PAD, D), jnp.float32)]
          + [pltpu.SemaphoreType.DMA] * (2 * NBUF + ISL)
      ),
  )(x, src_e, dst_e, w_e)


BR = 2000


def _mm_body(p_ref, w_ref, b_ref, o_ref):
  a = p_ref[0] + p_ref[1]
  o_ref[...] = lax.dot_general(
      a, w_ref[...], (((1,), (1,)), ((), ())),
      preferred_element_type=jnp.float32) + b_ref[...]


def _tc_linear(partials, W, b2):
  return pl.pallas_call(
      _mm_body,
      grid=(N // BR,),
      in_specs=[
          pl.BlockSpec((NC, BR, D), lambda i: (0, i, 0)),
          pl.BlockSpec((D, D), lambda i: (0, 0)),
          pl.BlockSpec((1, D), lambda i: (0, 0)),
      ],
      out_specs=pl.BlockSpec((BR, D), lambda i: (i, 0)),
      out_shape=jax.ShapeDtypeStruct((N, D), jnp.float32),
  )(partials, W, b2)


def kernel(x, edge_index, edge_weight, W, b):
  pad = E_PAD - E
  src = jnp.pad(edge_index[0].astype(jnp.int32), (0, pad))
  dst = jnp.pad(edge_index[1].astype(jnp.int32), (0, pad))
  wgt = jnp.pad(edge_weight.astype(jnp.float32), (0, pad))
  partials = _sc_aggregate(x, src, dst, wgt)
  return _tc_linear(partials, W, b.reshape(1, D))
